# Initial kernel scaffold; baseline (speedup 1.0000x reference)
import jax
import jax.numpy as jnp
from jax.experimental import pallas as pl


def kernel(input, emb_table, W1, b1, W2, b2):
    raise NotImplementedError("write your pallas kernel here")


# speedup vs baseline: 23.5443x; 23.5443x over previous
import functools

import jax
import jax.numpy as jnp
from jax import lax
from jax.experimental import pallas as pl
from jax.experimental.pallas import tpu as pltpu
from jax.experimental.pallas import tpu_sc as plsc

VOCAB = 10000
EMBED_DIM = 128
BATCH = 4096
HIST = 50

NUM_CORES = 2
NUM_SUBCORES = 16
LANES = 16
NW = NUM_CORES * NUM_SUBCORES
BPW = BATCH // NW
IPW = BPW * HIST
GROUPS = BPW // LANES


def _lut_body(table_ref, w1_ref, w2_ref, b1_ref, b2_ref, s_ref):
    f32 = jnp.float32
    hi = lax.Precision.HIGHEST
    v = lax.dot_general(w1_ref[...], w2_ref[...], (((0,), (1,)), ((), ())),
                        precision=hi, preferred_element_type=f32)
    s = lax.dot_general(table_ref[...], v, (((1,), (0,)), ((), ())),
                        precision=hi, preferred_element_type=f32)
    c = jnp.sum(b1_ref[...] * w2_ref[...]) + b2_ref[0, 0]
    s_ref[...] = (s + c) * (1.0 / HIST)


def _lut_call(table, w1, w2, b1r, b2r, interpret=False):
    return pl.pallas_call(
        _lut_body,
        out_shape=jax.ShapeDtypeStruct((VOCAB, 1), jnp.float32),
        interpret=interpret,
    )(table, w1, w2, b1r, b2r)


@functools.cache
def _get_bag_kernel():
    mesh = plsc.VectorSubcoreMesh(core_axis_name="c", subcore_axis_name="s",
                                  num_cores=NUM_CORES,
                                  num_subcores=NUM_SUBCORES)

    @functools.partial(
        pl.kernel,
        out_type=jax.ShapeDtypeStruct((BATCH,), jnp.float32),
        mesh=mesh,
        scratch_types=[
            pltpu.VMEM((VOCAB,), jnp.float32),
            pltpu.VMEM((IPW,), jnp.int32),
            pltpu.VMEM((BPW,), jnp.float32),
        ],
        compiler_params=pltpu.CompilerParams(needs_layout_passes=False),
    )
    def _bag_kernel(idx_hbm, lut_hbm, out_hbm, lut_v, idx_v, out_v):
        wid = lax.axis_index("s") * NUM_CORES + lax.axis_index("c")
        pltpu.sync_copy(lut_hbm, lut_v)
        pltpu.sync_copy(idx_hbm.at[pl.ds(wid * IPW, IPW)], idx_v)
        lane = lax.iota(jnp.int32, LANES)
        offs = lane * HIST
        for g in range(GROUPS):
            gbase = g * LANES * HIST

            def step(l, acc):
                pos = offs + (gbase + l)
                iv = plsc.load_gather(idx_v, [pos])
                return acc + plsc.load_gather(lut_v, [iv])

            acc = lax.fori_loop(0, HIST, step,
                                jnp.zeros((LANES,), jnp.float32))
            out_v[pl.ds(g * LANES, LANES)] = acc
        pltpu.sync_copy(out_v, out_hbm.at[pl.ds(wid * BPW, BPW)])

    return _bag_kernel


def kernel(input, emb_table, W1, b1, W2, b2):
    lut = _lut_call(emb_table, W1, W2, b1.reshape(1, EMBED_DIM),
                    b2.reshape(1, 1))
    out = _get_bag_kernel()(input.reshape(-1), lut.reshape(-1))
    return out.reshape(BATCH, 1)
